# Initial kernel scaffold; baseline (speedup 1.0000x reference)
import functools

import jax
import jax.numpy as jnp
from jax import lax
from jax.experimental import pallas as pl
from jax.experimental.pallas import tpu as pltpu
from jax.experimental.pallas import tpu_sc as plsc

N_NODES = 10000
N_EDGES = 320000
D = 128
G = 64
NC, NS, L = 2, 16, 16
NW = NC * NS
CHUNK = 128
EPW = 10240
NCH = EPW // CHUNK
EPAD = NW * EPW
NPAD = 10240
RPW = NPAD // NW
GP = 72
NPT = N_NODES // NS


def _mesh():
    return plsc.VectorSubcoreMesh(
        core_axis_name="c", subcore_axis_name="s", num_cores=NC, num_subcores=NS)


def _edge_body(x_hbm, src_hbm, dst_hbm, cci_hbm, out_hbm,
               acc_sh, src_v, dst_v, cci_v, rows_v, zbuf, sem):
    cid = lax.axis_index("c")
    sid = lax.axis_index("s")
    wid = sid * NC + cid

    zv = jnp.zeros((L,), jnp.float32)

    def zrow(r, _):
        for j in range(D // L):
            zbuf[r, pl.ds(j * L, L)] = zv
        return 0
    lax.fori_loop(0, 128, zrow, 0)
    for k in range(5):
        pltpu.sync_copy(zbuf.at[pl.ds(0, 125)],
                        acc_sh.at[pl.ds(sid * NPT + k * 125, 125)])
    plsc.subcore_barrier()

    pltpu.sync_copy(src_hbm.at[wid], src_v)
    pltpu.sync_copy(dst_hbm.at[wid], dst_v)
    pltpu.sync_copy(cci_hbm.at[wid], cci_v)

    def chunk_body(t, _):
        pltpu.async_copy(x_hbm.at[src_v.at[t]], rows_v, sem).wait()
        tvec = jnp.full((L,), t, jnp.int32)

        def row_body(r, _):
            c = plsc.load_gather(cci_v, [tvec, jnp.full((L,), r, jnp.int32)])
            for j in range(D // L):
                sl = pl.ds(j * L, L)
                rows_v[r, sl] = rows_v[r, sl] * c
            return 0
        lax.fori_loop(0, CHUNK, row_body, 0)
        pltpu.sync_copy(rows_v, acc_sh.at[dst_v.at[t]], add=True)
        return 0
    lax.fori_loop(0, NCH, chunk_body, 0)

    plsc.subcore_barrier()
    pltpu.sync_copy(acc_sh.at[pl.ds(sid * NPT, NPT)],
                    out_hbm.at[cid, pl.ds(sid * NPT, NPT)])
    @pl.when(sid == 0)
    def _():
        pltpu.sync_copy(zbuf.at[pl.ds(0, 128)],
                        out_hbm.at[cid, pl.ds(N_NODES, 128)])
        pltpu.sync_copy(zbuf.at[pl.ds(0, NPAD - N_NODES - 128)],
                        out_hbm.at[cid, pl.ds(N_NODES + 128, NPAD - N_NODES - 128)])


def _edge_pass(x, srcr, dstr, ccir):
    kfn = pl.kernel(
        _edge_body,
        out_type=jax.ShapeDtypeStruct((NC, NPAD, D), jnp.float32),
        mesh=_mesh(),
        scratch_types=[
            pltpu.VMEM_SHARED((N_NODES, D), jnp.float32),
            pltpu.VMEM((NCH, CHUNK), jnp.int32),
            pltpu.VMEM((NCH, CHUNK), jnp.int32),
            pltpu.VMEM((NCH, CHUNK), jnp.float32),
            pltpu.VMEM((CHUNK, D), jnp.float32),
            pltpu.VMEM((128, D), jnp.float32),
            pltpu.SemaphoreType.DMA,
        ],
    )
    return kfn(x, srcr, dstr, ccir)


def _layer_res_body(p_ref, w_ref, xp_ref, o_ref):
    acc = p_ref[0] + p_ref[1]
    h = jnp.dot(acc, w_ref[...], preferred_element_type=jnp.float32)
    o_ref[...] = jnp.maximum(h + xp_ref[...], 0.0)


def _layer_body(p_ref, w_ref, o_ref):
    acc = p_ref[0] + p_ref[1]
    h = jnp.dot(acc, w_ref[...], preferred_element_type=jnp.float32)
    o_ref[...] = jnp.maximum(h, 0.0)


def _layer(p, W, xprev):
    nb = 16
    rb = NPAD // nb
    in_specs = [
        pl.BlockSpec((NC, rb, D), lambda i: (0, i, 0)),
        pl.BlockSpec((D, D), lambda i: (0, 0)),
    ]
    args = [p, W]
    body = _layer_body
    if xprev is not None:
        in_specs.append(pl.BlockSpec((rb, D), lambda i: (i, 0)))
        args.append(xprev)
        body = _layer_res_body
    return pl.pallas_call(
        body,
        grid=(nb,),
        in_specs=in_specs,
        out_specs=pl.BlockSpec((rb, D), lambda i: (i, 0)),
        out_shape=jax.ShapeDtypeStruct((NPAD, D), jnp.float32),
    )(*args)


def _pool_body(x_hbm, bat_hbm, stats_hbm, cnt_hbm,
               xl_v, bat_v, sum_v, sq_v, mx_v, mn_v, cnt_v, sem):
    cid = lax.axis_index("c")
    sid = lax.axis_index("s")
    wid = sid * NC + cid

    zv = jnp.zeros((L,), jnp.float32)
    ninf = jnp.full((L,), -jnp.inf, jnp.float32)
    pinf = jnp.full((L,), jnp.inf, jnp.float32)

    def init_row(r, _):
        for j in range(D // L):
            sl = pl.ds(j * L, L)
            sum_v[r, sl] = zv
            sq_v[r, sl] = zv
            mx_v[r, sl] = ninf
            mn_v[r, sl] = pinf
        cnt_v[r, pl.ds(0, L)] = zv
        return 0
    lax.fori_loop(0, GP, init_row, 0)

    pltpu.sync_copy(x_hbm.at[pl.ds(wid * RPW, RPW)], xl_v)
    pltpu.sync_copy(bat_hbm.at[wid], bat_v)

    iota = lax.iota(jnp.int32, L)
    ones = jnp.full((L,), 1.0, jnp.float32)

    def row_body(r, _):
        g = plsc.load_gather(bat_v, [jnp.full((L,), r, jnp.int32)])
        plsc.addupdate_scatter(cnt_v, [g, iota], ones)
        for j in range(D // L):
            idx2 = iota + (j * L)
            xv = xl_v[r, pl.ds(j * L, L)]
            plsc.addupdate_scatter(sum_v, [g, idx2], xv)
            plsc.addupdate_scatter(sq_v, [g, idx2], xv * xv)
            cm = plsc.load_gather(mx_v, [g, idx2])
            plsc.store_scatter(mx_v, [g, idx2], jnp.maximum(cm, xv))
            cn = plsc.load_gather(mn_v, [g, idx2])
            plsc.store_scatter(mn_v, [g, idx2], jnp.minimum(cn, xv))
        return 0
    lax.fori_loop(0, RPW, row_body, 0)

    pltpu.sync_copy(sum_v, stats_hbm.at[wid, 0])
    pltpu.sync_copy(sq_v, stats_hbm.at[wid, 1])
    pltpu.sync_copy(mx_v, stats_hbm.at[wid, 2])
    pltpu.sync_copy(mn_v, stats_hbm.at[wid, 3])
    pltpu.sync_copy(cnt_v, cnt_hbm.at[wid])


def _pool(x2, batp):
    kfn = pl.kernel(
        _pool_body,
        out_type=(jax.ShapeDtypeStruct((NW, 4, GP, D), jnp.float32),
                  jax.ShapeDtypeStruct((NW, GP, L), jnp.float32)),
        mesh=_mesh(),
        scratch_types=[
            pltpu.VMEM((RPW, D), jnp.float32),
            pltpu.VMEM((RPW,), jnp.int32),
            pltpu.VMEM((GP, D), jnp.float32),
            pltpu.VMEM((GP, D), jnp.float32),
            pltpu.VMEM((GP, D), jnp.float32),
            pltpu.VMEM((GP, D), jnp.float32),
            pltpu.VMEM((GP, L), jnp.float32),
            pltpu.SemaphoreType.DMA,
        ],
    )
    return kfn(x2, batp)


def _head_body(stats_ref, cnt_ref, gf_ref,
               w1_ref, b1_ref, w2_ref, b2_ref, w3_ref, b3_ref, w4_ref, b4_ref,
               o_ref):
    s = stats_ref[...]
    sums = jnp.sum(s[:, 0], axis=0)[:G]
    sq = jnp.sum(s[:, 1], axis=0)[:G]
    mx = jnp.max(s[:, 2], axis=0)[:G]
    mn = jnp.min(s[:, 3], axis=0)[:G]
    cnt = jnp.sum(cnt_ref[...], axis=0)[:G, 0:1]
    cnt = jnp.maximum(cnt, 1.0)
    avg = sums / cnt
    var = jnp.maximum(sq / cnt - avg * avg, 0.0)
    std = jnp.sqrt(var + 1e-06)
    z = jnp.concatenate([avg, std, mx, mn, gf_ref[...][:, :4]], axis=1)
    z = jnp.maximum(jnp.dot(z, w1_ref[...], preferred_element_type=jnp.float32)
                    + b1_ref[...], 0.0)
    z = jnp.maximum(jnp.dot(z, w2_ref[...], preferred_element_type=jnp.float32)
                    + b2_ref[...], 0.0)
    z = jnp.maximum(jnp.dot(z, w3_ref[...], preferred_element_type=jnp.float32)
                    + b3_ref[...], 0.0)
    z = jnp.dot(z, w4_ref[...], preferred_element_type=jnp.float32) + b4_ref[...]
    half = z.shape[1] // 2
    o_ref[...] = jnp.concatenate([z[:, :half], jnp.square(z[:, half:])], axis=1)


def _head(stats, cnts, gf, fc1_w, fc1_b, fc2_w, fc2_b, fc3_w, fc3_b, fc4_w, fc4_b):
    return pl.pallas_call(
        _head_body,
        out_shape=jax.ShapeDtypeStruct((G, 2), jnp.float32),
    )(stats, cnts, gf,
      fc1_w, fc1_b.reshape(1, -1), fc2_w, fc2_b.reshape(1, -1),
      fc3_w, fc3_b.reshape(1, -1), fc4_w, fc4_b.reshape(1, -1))


def kernel(x_0, n0_to_0, cci_0_to_0, global_feature, batch_0,
           W1, W2, fc1_w, fc1_b, fc2_w, fc2_b, fc3_w, fc3_b, fc4_w, fc4_b):
    src = n0_to_0[0].astype(jnp.int32)
    dst = n0_to_0[1].astype(jnp.int32)
    cci = cci_0_to_0.astype(jnp.float32)

    npad_e = EPAD - N_EDGES
    pidx = (jnp.arange(npad_e, dtype=jnp.int32) * 13) % N_NODES
    srcr = jnp.concatenate([src, pidx]).reshape(NW, NCH, CHUNK)
    dstr = jnp.concatenate([dst, pidx]).reshape(NW, NCH, CHUNK)
    ccir = jnp.concatenate([cci, jnp.zeros((npad_e,), jnp.float32)]
                           ).reshape(NW, NCH, CHUNK)

    x0p = jnp.pad(x_0, ((0, NPAD - N_NODES), (0, 0)))
    batp = jnp.concatenate([batch_0.astype(jnp.int32),
                            jnp.full((NPAD - N_NODES,), G, jnp.int32)]
                           ).reshape(NW, RPW)

    p1 = _edge_pass(x0p, srcr, dstr, ccir)
    x1 = _layer(p1, W1, None)
    p2 = _edge_pass(x1, srcr, dstr, ccir)
    x2 = _layer(p2, W2, x1)
    stats, cnts = _pool(x2, batp)
    return _head(stats, cnts, global_feature,
                 fc1_w, fc1_b, fc2_w, fc2_b, fc3_w, fc3_b, fc4_w, fc4_b)


# speedup vs baseline: 6.8179x; 6.8179x over previous
import functools

import jax
import jax.numpy as jnp
from jax import lax
from jax.experimental import pallas as pl
from jax.experimental.pallas import tpu as pltpu
from jax.experimental.pallas import tpu_sc as plsc

N_NODES = 10000
N_EDGES = 320000
D = 128
G = 64
NC, NS, L = 2, 16, 16
NW = NC * NS
CHUNK = 128
EPW = 10240
NCH = EPW // CHUNK
EPAD = NW * EPW
NPAD = 10240
RPW = NPAD // NW
GP = 72
NPT = NPAD // NS


def _mesh():
    return plsc.VectorSubcoreMesh(
        core_axis_name="c", subcore_axis_name="s", num_cores=NC, num_subcores=NS)


def _edge_body(x_hbm, src_hbm, dst_hbm, cci_hbm, out_hbm,
               acc_sh, src_v, dst_v, cci_v, rows_v, sem):
    cid = lax.axis_index("c")
    sid = lax.axis_index("s")
    wid = sid * NC + cid

    zv = jnp.zeros((L,), jnp.float32)

    def zrow(r, _):
        for j in range(D // L):
            rows_v[r, pl.ds(j * L, L)] = zv
        return 0
    lax.fori_loop(0, CHUNK, zrow, 0)
    for k in range(NPT // CHUNK):
        pltpu.sync_copy(rows_v.at[pl.ds(0, CHUNK)],
                        acc_sh.at[pl.ds(sid * NPT + k * CHUNK, CHUNK)])
    plsc.subcore_barrier()

    pltpu.sync_copy(src_hbm.at[wid], src_v)
    pltpu.sync_copy(dst_hbm.at[wid], dst_v)
    pltpu.sync_copy(cci_hbm.at[wid], cci_v)

    iota = lax.iota(jnp.int32, L)

    def chunk_body(t, _):
        pltpu.async_copy(x_hbm.at[src_v.at[t]], rows_v, sem).wait()

        def grp_body(rg, _):
            cvec = cci_v[t, pl.ds(rg * L, L)]
            for rr in range(L):
                c = lax.broadcast(cvec[rr], (L,))
                r = rg * L + rr
                for j in range(D // L):
                    sl = pl.ds(j * L, L)
                    rows_v[r, sl] = rows_v[r, sl] * c
            return 0
        lax.fori_loop(0, CHUNK // L, grp_body, 0)
        pltpu.sync_copy(rows_v, acc_sh.at[dst_v.at[t]], add=True)
        return 0
    lax.fori_loop(0, NCH, chunk_body, 0)

    plsc.subcore_barrier()
    pltpu.sync_copy(acc_sh.at[pl.ds(sid * NPT, NPT)],
                    out_hbm.at[cid, pl.ds(sid * NPT, NPT)])


def _edge_pass(x, srcr, dstr, ccir):
    kfn = pl.kernel(
        _edge_body,
        out_type=jax.ShapeDtypeStruct((NC, NPAD, D), jnp.float32),
        mesh=_mesh(),
        scratch_types=[
            pltpu.VMEM_SHARED((NPAD, D), jnp.float32),
            pltpu.VMEM((NCH, CHUNK), jnp.int32),
            pltpu.VMEM((NCH, CHUNK), jnp.int32),
            pltpu.VMEM((NCH, CHUNK), jnp.float32),
            pltpu.VMEM((CHUNK, D), jnp.float32),
            pltpu.SemaphoreType.DMA,
        ],
    )
    return kfn(x, srcr, dstr, ccir)


def _layer_res_body(p_ref, w_ref, xp_ref, o_ref):
    acc = p_ref[0] + p_ref[1]
    h = jnp.dot(acc, w_ref[...], preferred_element_type=jnp.float32)
    o_ref[...] = jnp.maximum(h + xp_ref[...], 0.0)


def _layer_body(p_ref, w_ref, o_ref):
    acc = p_ref[0] + p_ref[1]
    h = jnp.dot(acc, w_ref[...], preferred_element_type=jnp.float32)
    o_ref[...] = jnp.maximum(h, 0.0)


def _layer(p, W, xprev):
    nb = 16
    rb = NPAD // nb
    in_specs = [
        pl.BlockSpec((NC, rb, D), lambda i: (0, i, 0)),
        pl.BlockSpec((D, D), lambda i: (0, 0)),
    ]
    args = [p, W]
    body = _layer_body
    if xprev is not None:
        in_specs.append(pl.BlockSpec((rb, D), lambda i: (i, 0)))
        args.append(xprev)
        body = _layer_res_body
    return pl.pallas_call(
        body,
        grid=(nb,),
        in_specs=in_specs,
        out_specs=pl.BlockSpec((rb, D), lambda i: (i, 0)),
        out_shape=jax.ShapeDtypeStruct((NPAD, D), jnp.float32),
    )(*args)


def _pool_body(x_hbm, bat_hbm, stats_hbm, cnt_hbm,
               xl_v, bat_v, sum_v, sq_v, mx_v, mn_v, cnt_v, sem):
    cid = lax.axis_index("c")
    sid = lax.axis_index("s")
    wid = sid * NC + cid

    zv = jnp.zeros((L,), jnp.float32)
    ninf = jnp.full((L,), -jnp.inf, jnp.float32)
    pinf = jnp.full((L,), jnp.inf, jnp.float32)

    def init_row(r, _):
        for j in range(D // L):
            sl = pl.ds(j * L, L)
            sum_v[r, sl] = zv
            sq_v[r, sl] = zv
            mx_v[r, sl] = ninf
            mn_v[r, sl] = pinf
        cnt_v[r, pl.ds(0, L)] = zv
        return 0
    lax.fori_loop(0, GP, init_row, 0)

    pltpu.sync_copy(x_hbm.at[pl.ds(wid * RPW, RPW)], xl_v)
    pltpu.sync_copy(bat_hbm.at[wid], bat_v)

    iota = lax.iota(jnp.int32, L)

    def grp_body(rg, _):
        bvec = bat_v[pl.ds(rg * L, L)]
        for rr in range(L):
            g = bvec[rr]
            r = rg * L + rr
            cs = pl.ds(0, L)
            cnt_v[g, cs] = cnt_v[g, cs] + 1.0
            for j in range(D // L):
                sl = pl.ds(j * L, L)
                xv = xl_v[r, sl]
                sum_v[g, sl] = sum_v[g, sl] + xv
                sq_v[g, sl] = sq_v[g, sl] + xv * xv
                mx_v[g, sl] = jnp.maximum(mx_v[g, sl], xv)
                mn_v[g, sl] = jnp.minimum(mn_v[g, sl], xv)
        return 0
    lax.fori_loop(0, RPW // L, grp_body, 0)

    pltpu.sync_copy(sum_v, stats_hbm.at[wid, 0])
    pltpu.sync_copy(sq_v, stats_hbm.at[wid, 1])
    pltpu.sync_copy(mx_v, stats_hbm.at[wid, 2])
    pltpu.sync_copy(mn_v, stats_hbm.at[wid, 3])
    pltpu.sync_copy(cnt_v, cnt_hbm.at[wid])


def _pool(x2, batp):
    kfn = pl.kernel(
        _pool_body,
        out_type=(jax.ShapeDtypeStruct((NW, 4, GP, D), jnp.float32),
                  jax.ShapeDtypeStruct((NW, GP, L), jnp.float32)),
        mesh=_mesh(),
        scratch_types=[
            pltpu.VMEM((RPW, D), jnp.float32),
            pltpu.VMEM((RPW,), jnp.int32),
            pltpu.VMEM((GP, D), jnp.float32),
            pltpu.VMEM((GP, D), jnp.float32),
            pltpu.VMEM((GP, D), jnp.float32),
            pltpu.VMEM((GP, D), jnp.float32),
            pltpu.VMEM((GP, L), jnp.float32),
            pltpu.SemaphoreType.DMA,
        ],
    )
    return kfn(x2, batp)


def _head_body(stats_ref, cnt_ref, gf_ref,
               w1_ref, b1_ref, w2_ref, b2_ref, w3_ref, b3_ref, w4_ref, b4_ref,
               o_ref):
    s = stats_ref[...]
    sums = jnp.sum(s[:, 0], axis=0)[:G]
    sq = jnp.sum(s[:, 1], axis=0)[:G]
    mx = jnp.max(s[:, 2], axis=0)[:G]
    mn = jnp.min(s[:, 3], axis=0)[:G]
    cnt = jnp.sum(cnt_ref[...], axis=0)[:G, 0:1]
    cnt = jnp.maximum(cnt, 1.0)
    avg = sums / cnt
    var = jnp.maximum(sq / cnt - avg * avg, 0.0)
    std = jnp.sqrt(var + 1e-06)
    z = jnp.concatenate([avg, std, mx, mn, gf_ref[...][:, :4]], axis=1)
    z = jnp.maximum(jnp.dot(z, w1_ref[...], preferred_element_type=jnp.float32)
                    + b1_ref[...], 0.0)
    z = jnp.maximum(jnp.dot(z, w2_ref[...], preferred_element_type=jnp.float32)
                    + b2_ref[...], 0.0)
    z = jnp.maximum(jnp.dot(z, w3_ref[...], preferred_element_type=jnp.float32)
                    + b3_ref[...], 0.0)
    z = jnp.dot(z, w4_ref[...], preferred_element_type=jnp.float32) + b4_ref[...]
    half = z.shape[1] // 2
    o_ref[...] = jnp.concatenate([z[:, :half], jnp.square(z[:, half:])], axis=1)


def _head(stats, cnts, gf, fc1_w, fc1_b, fc2_w, fc2_b, fc3_w, fc3_b, fc4_w, fc4_b):
    return pl.pallas_call(
        _head_body,
        out_shape=jax.ShapeDtypeStruct((G, 2), jnp.float32),
    )(stats, cnts, gf,
      fc1_w, fc1_b.reshape(1, -1), fc2_w, fc2_b.reshape(1, -1),
      fc3_w, fc3_b.reshape(1, -1), fc4_w, fc4_b.reshape(1, -1))


def kernel(x_0, n0_to_0, cci_0_to_0, global_feature, batch_0,
           W1, W2, fc1_w, fc1_b, fc2_w, fc2_b, fc3_w, fc3_b, fc4_w, fc4_b):
    src = n0_to_0[0].astype(jnp.int32)
    dst = n0_to_0[1].astype(jnp.int32)
    cci = cci_0_to_0.astype(jnp.float32)

    npad_e = EPAD - N_EDGES
    pidx = (jnp.arange(npad_e, dtype=jnp.int32) * 13) % N_NODES
    srcr = jnp.concatenate([src, pidx]).reshape(NW, NCH, CHUNK)
    dstr = jnp.concatenate([dst, pidx]).reshape(NW, NCH, CHUNK)
    ccir = jnp.concatenate([cci, jnp.zeros((npad_e,), jnp.float32)]
                           ).reshape(NW, NCH, CHUNK)

    x0p = jnp.pad(x_0, ((0, NPAD - N_NODES), (0, 0)))
    batp = jnp.concatenate([batch_0.astype(jnp.int32),
                            jnp.full((NPAD - N_NODES,), G, jnp.int32)]
                           ).reshape(NW, RPW)

    p1 = _edge_pass(x0p, srcr, dstr, ccir)
    x1 = _layer(p1, W1, None)
    p2 = _edge_pass(x1, srcr, dstr, ccir)
    x2 = _layer(p2, W2, x1)
    stats, cnts = _pool(x2, batp)
    return _head(stats, cnts, global_feature,
                 fc1_w, fc1_b, fc2_w, fc2_b, fc3_w, fc3_b, fc4_w, fc4_b)


# speedup vs baseline: 8.2899x; 1.2159x over previous
import functools

import jax
import jax.numpy as jnp
from jax import lax
from jax.experimental import pallas as pl
from jax.experimental.pallas import tpu as pltpu
from jax.experimental.pallas import tpu_sc as plsc

N_NODES = 10000
N_EDGES = 320000
D = 128
G = 64
NC, NS, L = 2, 16, 16
NW = NC * NS
CHUNK = 64
NCH = 159
EPW = NCH * CHUNK
NBUF = 3
EPAD = NW * EPW
NPAD = 10240
RPW = NPAD // NW
GP = 72
NPT = NPAD // NS


def _mesh():
    return plsc.VectorSubcoreMesh(
        core_axis_name="c", subcore_axis_name="s", num_cores=NC, num_subcores=NS)


def _edge_body(x_hbm, sd_hbm, cci_hbm, out_hbm,
               acc_sh, cci_v, sd_st, rows_v,
               g0, g1, g2, s0, s1, s2):
    cid = lax.axis_index("c")
    sid = lax.axis_index("s")
    wid = sid * NC + cid
    gsem = [g0, g1, g2]
    ssem = [s0, s1, s2]

    zv = jnp.zeros((L,), jnp.float32)

    def zrow(r, _):
        for j in range(D // L):
            rows_v[0, r, pl.ds(j * L, L)] = zv
        return 0
    lax.fori_loop(0, CHUNK, zrow, 0)
    for k in range(NPT // CHUNK):
        pltpu.sync_copy(rows_v.at[0],
                        acc_sh.at[pl.ds(sid * NPT + k * CHUNK, CHUNK)])
    plsc.subcore_barrier()

    pltpu.sync_copy(cci_hbm.at[wid], cci_v)

    def stage(f, slot):
        pltpu.sync_copy(sd_hbm.at[wid, f], sd_st.at[slot])

    def issue_gather(f, slot):
        pltpu.async_copy(x_hbm.at[sd_st.at[slot, 0]], rows_v.at[slot],
                         gsem[slot])

    def issue_scatter(t, slot):
        pltpu.async_copy(rows_v.at[slot], acc_sh.at[sd_st.at[slot, 1]],
                         ssem[slot], add=True)

    def scale(t, slot):
        def grp_body(rg, _):
            cvec = cci_v[t, pl.ds(rg * L, L)]
            for rr in range(L):
                c = lax.broadcast(cvec[rr], (L,))
                r = rg * L + rr
                for j in range(D // L):
                    sl = pl.ds(j * L, L)
                    rows_v[slot, r, sl] = rows_v[slot, r, sl] * c
            return 0
        lax.fori_loop(0, CHUNK // L, grp_body, 0)

    def wait_g(slot):
        pltpu.make_async_copy(x_hbm.at[sd_st.at[slot, 0]], rows_v.at[slot],
                              gsem[slot]).wait()

    def wait_s(slot):
        pltpu.make_async_copy(rows_v.at[slot], acc_sh.at[sd_st.at[slot, 1]],
                              ssem[slot]).wait()

    for b in range(NBUF - 1):
        stage(b, b)
        issue_gather(b, b)

    def step(t, b, prefetch, first):
        wait_g(b)
        scale(t, b)
        issue_scatter(t, b)
        if prefetch:
            b2 = (b + NBUF - 1) % NBUF
            if not first:
                wait_s(b2)
            stage(t + NBUF - 1, b2)
            issue_gather(t + NBUF - 1, b2)

    for b in range(NBUF):
        step(b, b, True, b == 0)

    NU = NCH // NBUF

    def u_body(u, _):
        for b in range(NBUF):
            step(u * NBUF + b, b, True, False)
        return 0
    lax.fori_loop(1, NU - 1, u_body, 0)

    step((NU - 1) * NBUF, 0, True, False)
    for b in range(1, NBUF):
        t = (NU - 1) * NBUF + b
        wait_g(b)
        scale(t, b)
        issue_scatter(t, b)

    for b in range(NBUF):
        wait_s(b)

    plsc.subcore_barrier()
    pltpu.sync_copy(acc_sh.at[pl.ds(sid * NPT, NPT)],
                    out_hbm.at[cid, pl.ds(sid * NPT, NPT)])


def _edge_pass(x, sd, ccir):
    kfn = pl.kernel(
        _edge_body,
        out_type=jax.ShapeDtypeStruct((NC, NPAD, D), jnp.float32),
        mesh=_mesh(),
        scratch_types=[
            pltpu.VMEM_SHARED((NPAD, D), jnp.float32),
            pltpu.VMEM((NCH, CHUNK), jnp.float32),
            pltpu.VMEM((NBUF, 2, CHUNK), jnp.int32),
            pltpu.VMEM((NBUF, CHUNK, D), jnp.float32),
            pltpu.SemaphoreType.DMA,
            pltpu.SemaphoreType.DMA,
            pltpu.SemaphoreType.DMA,
            pltpu.SemaphoreType.DMA,
            pltpu.SemaphoreType.DMA,
            pltpu.SemaphoreType.DMA,
        ],
    )
    return kfn(x, sd, ccir)


def _layer_res_body(p_ref, w_ref, xp_ref, o_ref):
    acc = p_ref[0] + p_ref[1]
    h = jnp.dot(acc, w_ref[...], preferred_element_type=jnp.float32)
    o_ref[...] = jnp.maximum(h + xp_ref[...], 0.0)


def _layer_body(p_ref, w_ref, o_ref):
    acc = p_ref[0] + p_ref[1]
    h = jnp.dot(acc, w_ref[...], preferred_element_type=jnp.float32)
    o_ref[...] = jnp.maximum(h, 0.0)


def _layer(p, W, xprev):
    nb = 16
    rb = NPAD // nb
    in_specs = [
        pl.BlockSpec((NC, rb, D), lambda i: (0, i, 0)),
        pl.BlockSpec((D, D), lambda i: (0, 0)),
    ]
    args = [p, W]
    body = _layer_body
    if xprev is not None:
        in_specs.append(pl.BlockSpec((rb, D), lambda i: (i, 0)))
        args.append(xprev)
        body = _layer_res_body
    return pl.pallas_call(
        body,
        grid=(nb,),
        in_specs=in_specs,
        out_specs=pl.BlockSpec((rb, D), lambda i: (i, 0)),
        out_shape=jax.ShapeDtypeStruct((NPAD, D), jnp.float32),
    )(*args)


def _pool_body(x_hbm, bat_hbm, stats_hbm, cnt_hbm,
               xl_v, bat_v, sum_v, sq_v, mx_v, mn_v, cnt_v, sem):
    cid = lax.axis_index("c")
    sid = lax.axis_index("s")
    wid = sid * NC + cid

    zv = jnp.zeros((L,), jnp.float32)
    ninf = jnp.full((L,), -jnp.inf, jnp.float32)
    pinf = jnp.full((L,), jnp.inf, jnp.float32)

    def init_row(r, _):
        for j in range(D // L):
            sl = pl.ds(j * L, L)
            sum_v[r, sl] = zv
            sq_v[r, sl] = zv
            mx_v[r, sl] = ninf
            mn_v[r, sl] = pinf
        cnt_v[r, pl.ds(0, L)] = zv
        return 0
    lax.fori_loop(0, GP, init_row, 0)

    pltpu.sync_copy(x_hbm.at[pl.ds(wid * RPW, RPW)], xl_v)
    pltpu.sync_copy(bat_hbm.at[wid], bat_v)

    iota = lax.iota(jnp.int32, L)

    def grp_body(rg, _):
        bvec = bat_v[pl.ds(rg * L, L)]
        for rr in range(L):
            g = bvec[rr]
            r = rg * L + rr
            cs = pl.ds(0, L)
            cnt_v[g, cs] = cnt_v[g, cs] + 1.0
            for j in range(D // L):
                sl = pl.ds(j * L, L)
                xv = xl_v[r, sl]
                sum_v[g, sl] = sum_v[g, sl] + xv
                sq_v[g, sl] = sq_v[g, sl] + xv * xv
                mx_v[g, sl] = jnp.maximum(mx_v[g, sl], xv)
                mn_v[g, sl] = jnp.minimum(mn_v[g, sl], xv)
        return 0
    lax.fori_loop(0, RPW // L, grp_body, 0)

    pltpu.sync_copy(sum_v, stats_hbm.at[wid, 0])
    pltpu.sync_copy(sq_v, stats_hbm.at[wid, 1])
    pltpu.sync_copy(mx_v, stats_hbm.at[wid, 2])
    pltpu.sync_copy(mn_v, stats_hbm.at[wid, 3])
    pltpu.sync_copy(cnt_v, cnt_hbm.at[wid])


def _pool(x2, batp):
    kfn = pl.kernel(
        _pool_body,
        out_type=(jax.ShapeDtypeStruct((NW, 4, GP, D), jnp.float32),
                  jax.ShapeDtypeStruct((NW, GP, L), jnp.float32)),
        mesh=_mesh(),
        scratch_types=[
            pltpu.VMEM((RPW, D), jnp.float32),
            pltpu.VMEM((RPW,), jnp.int32),
            pltpu.VMEM((GP, D), jnp.float32),
            pltpu.VMEM((GP, D), jnp.float32),
            pltpu.VMEM((GP, D), jnp.float32),
            pltpu.VMEM((GP, D), jnp.float32),
            pltpu.VMEM((GP, L), jnp.float32),
            pltpu.SemaphoreType.DMA,
        ],
    )
    return kfn(x2, batp)


def _head_body(stats_ref, cnt_ref, gf_ref,
               w1_ref, b1_ref, w2_ref, b2_ref, w3_ref, b3_ref, w4_ref, b4_ref,
               o_ref):
    s = stats_ref[...]
    sums = jnp.sum(s[:, 0], axis=0)[:G]
    sq = jnp.sum(s[:, 1], axis=0)[:G]
    mx = jnp.max(s[:, 2], axis=0)[:G]
    mn = jnp.min(s[:, 3], axis=0)[:G]
    cnt = jnp.sum(cnt_ref[...], axis=0)[:G, 0:1]
    cnt = jnp.maximum(cnt, 1.0)
    avg = sums / cnt
    var = jnp.maximum(sq / cnt - avg * avg, 0.0)
    std = jnp.sqrt(var + 1e-06)
    z = jnp.concatenate([avg, std, mx, mn, gf_ref[...][:, :4]], axis=1)
    z = jnp.maximum(jnp.dot(z, w1_ref[...], preferred_element_type=jnp.float32)
                    + b1_ref[...], 0.0)
    z = jnp.maximum(jnp.dot(z, w2_ref[...], preferred_element_type=jnp.float32)
                    + b2_ref[...], 0.0)
    z = jnp.maximum(jnp.dot(z, w3_ref[...], preferred_element_type=jnp.float32)
                    + b3_ref[...], 0.0)
    z = jnp.dot(z, w4_ref[...], preferred_element_type=jnp.float32) + b4_ref[...]
    half = z.shape[1] // 2
    o_ref[...] = jnp.concatenate([z[:, :half], jnp.square(z[:, half:])], axis=1)


def _head(stats, cnts, gf, fc1_w, fc1_b, fc2_w, fc2_b, fc3_w, fc3_b, fc4_w, fc4_b):
    return pl.pallas_call(
        _head_body,
        out_shape=jax.ShapeDtypeStruct((G, 2), jnp.float32),
    )(stats, cnts, gf,
      fc1_w, fc1_b.reshape(1, -1), fc2_w, fc2_b.reshape(1, -1),
      fc3_w, fc3_b.reshape(1, -1), fc4_w, fc4_b.reshape(1, -1))


def kernel(x_0, n0_to_0, cci_0_to_0, global_feature, batch_0,
           W1, W2, fc1_w, fc1_b, fc2_w, fc2_b, fc3_w, fc3_b, fc4_w, fc4_b):
    src = n0_to_0[0].astype(jnp.int32)
    dst = n0_to_0[1].astype(jnp.int32)
    cci = cci_0_to_0.astype(jnp.float32)

    npad_e = EPAD - N_EDGES
    pidx = (jnp.arange(npad_e, dtype=jnp.int32) * 13) % N_NODES
    srcr = jnp.concatenate([src, pidx]).reshape(NW, NCH, CHUNK)
    dstr = jnp.concatenate([dst, pidx]).reshape(NW, NCH, CHUNK)
    sd = jnp.stack([srcr, dstr], axis=2)
    ccir = jnp.concatenate([cci, jnp.zeros((npad_e,), jnp.float32)]
                           ).reshape(NW, NCH, CHUNK)

    x0p = jnp.pad(x_0, ((0, NPAD - N_NODES), (0, 0)))
    batp = jnp.concatenate([batch_0.astype(jnp.int32),
                            jnp.full((NPAD - N_NODES,), G, jnp.int32)]
                           ).reshape(NW, RPW)

    p1 = _edge_pass(x0p, sd, ccir)
    x1 = _layer(p1, W1, None)
    p2 = _edge_pass(x1, sd, ccir)
    x2 = _layer(p2, W2, x1)
    stats, cnts = _pool(x2, batp)
    return _head(stats, cnts, global_feature,
                 fc1_w, fc1_b, fc2_w, fc2_b, fc3_w, fc3_b, fc4_w, fc4_b)
